# Initial kernel scaffold; baseline (speedup 1.0000x reference)
import jax
import jax.numpy as jnp
from jax.experimental import pallas as pl


def kernel(x, W_g, W_s):
    raise NotImplementedError("write your pallas kernel here")


# speedup vs baseline: 2.8845x; 2.8845x over previous
import functools

import jax
import jax.numpy as jnp
from jax import lax
from jax.experimental import pallas as pl
from jax.experimental.pallas import tpu as pltpu
from jax.experimental.pallas import tpu_sc as plsc

_VOCAB = 100000
_D = 64
_S = 8
_B = 4096
_SEQ = 52
_L = 16
_NC = 2
_NS = 16
_NW = _NC * _NS
_BPW = _B // _NW
_KD = _D // _L


@functools.partial(
    pl.kernel,
    mesh=plsc.VectorSubcoreMesh(core_axis_name="c", subcore_axis_name="s"),
    compiler_params=pltpu.CompilerParams(use_tc_tiling_on_sc=False),
    out_type=jax.ShapeDtypeStruct((_NW, _S, _L), jnp.float32),
    scratch_types=[
        pltpu.VMEM((_BPW,), jnp.int32),
        pltpu.VMEM((_BPW, _D), jnp.float32),
        pltpu.VMEM((_BPW, _D), jnp.float32),
        pltpu.VMEM((_BPW, _S * _D), jnp.float32),
        pltpu.VMEM((_S, _L), jnp.float32),
        pltpu.SemaphoreType.DMA,
    ],
)
def _sense_partials(xT_hbm, wg_hbm, ws_hbm, out_hbm,
                    idx_v, rows_v, acc_v, ws_v, part_v, sem):
    wid = lax.axis_index("s") * _NC + lax.axis_index("c")
    base = wid * _BPW

    zeros = jnp.zeros((_L,), jnp.float32)

    def zero_body(i, carry):
        for k in range(_KD):
            acc_v[i, pl.ds(k * _L, _L)] = zeros
        return carry

    lax.fori_loop(0, _BPW, zero_body, 0)

    def col_body(c, carry):
        pltpu.sync_copy(xT_hbm.at[c, pl.ds(base, _BPW)], idx_v)
        pltpu.async_copy(wg_hbm.at[idx_v], rows_v, sem).wait()

        def acc_body(i, inner):
            for k in range(_KD):
                sl = pl.ds(k * _L, _L)
                plsc.addupdate(acc_v.at[i, sl], rows_v[i, sl])
            return inner

        lax.fori_loop(0, _BPW, acc_body, 0)
        return carry

    lax.fori_loop(2, _SEQ, col_body, 0)

    pltpu.sync_copy(xT_hbm.at[0, pl.ds(base, _BPW)], idx_v)
    pltpu.async_copy(ws_hbm.at[idx_v], ws_v, sem).wait()

    def score_body(i, accs):
        ctx = [acc_v[i, pl.ds(k * _L, _L)] for k in range(_KD)]
        out = []
        for s in range(_S):
            a = accs[s]
            for k in range(_KD):
                a = a + ws_v[i, pl.ds(s * _D + k * _L, _L)] * ctx[k]
            out.append(a)
        return tuple(out)

    accs = lax.fori_loop(0, _BPW, score_body,
                         tuple(zeros for _ in range(_S)))
    for s in range(_S):
        part_v[s, :] = accs[s]
    pltpu.sync_copy(part_v, out_hbm.at[wid])


@jax.jit
def kernel(x, W_g, W_s):
    xT = x.T
    ws2 = W_s.reshape(_VOCAB, _S * _D)
    partials = _sense_partials(xT, W_g, ws2)
    return jax.nn.sigmoid(jnp.sum(partials, axis=(0, 2)))


# speedup vs baseline: 3.3773x; 1.1708x over previous
import functools

import jax
import jax.numpy as jnp
from jax import lax
from jax.experimental import pallas as pl
from jax.experimental.pallas import tpu as pltpu
from jax.experimental.pallas import tpu_sc as plsc

_VOCAB = 100000
_D = 64
_S = 8
_B = 4096
_SEQ = 52
_L = 16
_NC = 2
_NS = 16
_NW = _NC * _NS
_BPW = _B // _NW
_KD = _D // _L
_NBUF = 3


@functools.partial(
    pl.kernel,
    mesh=plsc.VectorSubcoreMesh(core_axis_name="c", subcore_axis_name="s"),
    compiler_params=pltpu.CompilerParams(use_tc_tiling_on_sc=False),
    out_type=jax.ShapeDtypeStruct((_NW, _S, _L), jnp.float32),
    scratch_types=[
        pltpu.VMEM((_SEQ, _BPW), jnp.int32),
        pltpu.VMEM((_NBUF, _BPW, _D), jnp.float32),
        pltpu.VMEM((_BPW, _D), jnp.float32),
        pltpu.VMEM((_BPW, _S * _D), jnp.float32),
        pltpu.VMEM((_S, _L), jnp.float32),
        pltpu.SemaphoreType.DMA,
        pltpu.SemaphoreType.DMA,
        pltpu.SemaphoreType.DMA,
        pltpu.SemaphoreType.DMA,
    ],
)
def _sense_partials(xT_hbm, wg_hbm, ws_hbm, out_hbm,
                    x_v, rows_v, acc_v, ws_v, part_v,
                    sem_ws, sem0, sem1, sem2):
    wid = lax.axis_index("s") * _NC + lax.axis_index("c")
    sems = (sem0, sem1, sem2)

    pltpu.sync_copy(xT_hbm.at[wid], x_v)

    pltpu.async_copy(ws_hbm.at[x_v.at[0]], ws_v, sem_ws)

    def start_col(c, buf):
        pltpu.async_copy(wg_hbm.at[x_v.at[c]], rows_v.at[buf], sems[buf])

    def wait_col(c, buf):
        pltpu.make_async_copy(
            wg_hbm.at[x_v.at[c]], rows_v.at[buf], sems[buf]).wait()

    def acc_col(buf, first):
        def body(i, carry):
            for k in range(_KD):
                sl = pl.ds(k * _L, _L)
                v = rows_v[buf, i, sl]
                if first:
                    acc_v[i, sl] = v
                else:
                    plsc.addupdate(acc_v.at[i, sl], v)
            return carry
        lax.fori_loop(0, _BPW, body, 0, unroll=4)

    for t in range(_NBUF):
        start_col(2 + t, t)

    wait_col(2, 0)
    acc_col(0, first=True)
    start_col(5, 0)

    def ring_body(j, carry):
        c0 = 3 + 3 * j
        for t in range(3):
            buf = (1 + t) % _NBUF
            c = c0 + t
            wait_col(c, buf)
            acc_col(buf, first=False)

            @pl.when(c + _NBUF < _SEQ)
            def _():
                start_col(c + _NBUF, buf)
        return carry

    lax.fori_loop(0, 16, ring_body, 0)

    wait_col(51, 1)
    acc_col(1, first=False)

    pltpu.make_async_copy(ws_hbm.at[x_v.at[0]], ws_v, sem_ws).wait()

    zeros = jnp.zeros((_L,), jnp.float32)

    def score_body(i, accs):
        ctx = [acc_v[i, pl.ds(k * _L, _L)] for k in range(_KD)]
        out = []
        for s in range(_S):
            a = accs[s]
            for k in range(_KD):
                a = a + ws_v[i, pl.ds(s * _D + k * _L, _L)] * ctx[k]
            out.append(a)
        return tuple(out)

    accs = lax.fori_loop(0, _BPW, score_body,
                         tuple(zeros for _ in range(_S)))
    for s in range(_S):
        part_v[s, :] = accs[s]
    pltpu.sync_copy(part_v, out_hbm.at[wid])


@jax.jit
def kernel(x, W_g, W_s):
    xT = x.T.reshape(_SEQ, _NW, _BPW).transpose(1, 0, 2)
    ws2 = W_s.reshape(_VOCAB, _S * _D)
    partials = _sense_partials(xT, W_g, ws2)
    return jax.nn.sigmoid(jnp.sum(partials, axis=(0, 2)))
